# baseline (speedup 1.0000x reference)
import functools
import math

import jax
import jax.numpy as jnp
from jax import lax
from jax.experimental import pallas as pl
from jax.experimental.pallas import tpu as pltpu
from jax.experimental.pallas import tpu_sc as plsc

N = 1_000_000
NIMG = 10_000
DM = 10
MC = 4
HMAX = 32
NU = HMAX ** 3
LOG2PI = float(math.log(2.0 * math.pi))

BN = 8000
GRID = N // BN
CH = 1600
NCHUNK = N // CH
NW = 32
KMAX = -(-NCHUNK // NW)


def _stats(md_ref, io_ref, msum_ref, msq_ref, isq_ref):
    @pl.when(pl.program_id(0) == 0)
    def _():
        msum_ref[...] = jnp.zeros_like(msum_ref)
        msq_ref[...] = jnp.zeros_like(msq_ref)
        isq_ref[...] = jnp.zeros_like(isq_ref)

    m = md_ref[...]
    msum_ref[...] += m.sum(axis=0, keepdims=True)
    msq_ref[...] += (m * m).sum(axis=0, keepdims=True)
    io = io_ref[...]
    isq_ref[...] += jnp.sum(io * io).reshape(1, 1)


def _post(ql_ref, qsr_ref, ep_ref, zt_ref, kl_ref):
    @pl.when(pl.program_id(0) == 0)
    def _():
        kl_ref[...] = jnp.zeros_like(kl_ref)

    ql = ql_ref[...]
    qs = jax.nn.softplus(qsr_ref[...]) + 1e-6
    ep = ep_ref[...]
    z = ql + qs * ep
    zt_ref[...] = jnp.concatenate([z, jnp.zeros((z.shape[0], 12), jnp.float32)], axis=1)
    kl_ref[...] += jnp.sum(0.5 * z * z - 0.5 * ep * ep - jnp.log(qs)).reshape(1, 1)


def _lin(md_ref, sio_ref, wl_ref, ws_ref, cst_ref,
         loc_ref, sig_ref, c4_ref, skl_ref):
    @pl.when(pl.program_id(0) == 0)
    def _():
        skl_ref[...] = jnp.zeros_like(skl_ref)

    m = md_ref[...]
    wl = wl_ref[...]
    ws = ws_ref[...]
    cst = cst_ref[...]
    loc = (m * wl).sum(axis=1) + cst[0, 0]
    sig = jax.nn.softplus((m * ws).sum(axis=1) + cst[0, 1]) + 1e-6
    loc_ref[...] = loc.reshape(1, 1, BN)
    sig_ref[...] = sig.reshape(1, 1, BN)
    sio = sio_ref[...]
    c4_ref[...] = 4.0 * (cst[0, 2] - jnp.log(sio))
    skl_ref[...] += jnp.sum(0.5 * (loc * loc + sig * sig - 2.0 * jnp.log(sig) - 1.0)).reshape(1, 1)


def _comb(llp_ref, cnp_ref, out_ref):
    s = llp_ref[...].sum(axis=0)
    c = jnp.maximum(cnp_ref[...].sum(axis=0), 1.0)
    out_ref[...] = (-(s / c) * (1.0 / MC)).reshape(1, NIMG)


def _take16(x, idx):
    dnums = lax.GatherDimensionNumbers(
        offset_dims=(), collapsed_slice_dims=(0,), start_index_map=(0,))
    return lax.gather(x, idx[:, None], dnums, (1,),
                      mode=lax.GatherScatterMode.PROMISE_IN_BOUNDS)


def _sc_body(loc_h, sig_h, io_h, sg_h, c4_h, img_h, hkl_h, eps_h, zt_h, scal_h,
             ipred_h, llp_h, cnp_h,
             vloc, vsig, vio, vsg, vc4, vimg, vhkl, veps, vidx, fbuf, vipred,
             acc_ll, acc_cn, scalv, sem):
    wid = lax.axis_index("s") * 2 + lax.axis_index("c")
    iota = lax.iota(jnp.int32, 16)
    zeros16 = jnp.zeros((16,), jnp.float32)

    def _zinit(i, carry):
        acc_ll[pl.ds(i * 16, 16)] = zeros16
        acc_cn[pl.ds(i * 16, 16)] = zeros16
        return carry
    lax.fori_loop(0, NIMG // 16, _zinit, None)

    pltpu.sync_copy(scal_h, scalv)
    iv = scalv[...]

    def _chunk(k, carry):
        c = wid + NW * k

        @pl.when(c < NCHUNK)
        def _():
            base = c * CH
            pltpu.sync_copy(loc_h.at[pl.ds(base, CH)], vloc)
            pltpu.sync_copy(sig_h.at[pl.ds(base, CH)], vsig)
            pltpu.sync_copy(io_h.at[pl.ds(base, CH)], vio)
            pltpu.sync_copy(sg_h.at[pl.ds(base, CH)], vsg)
            pltpu.sync_copy(c4_h.at[pl.ds(base, CH)], vc4)
            pltpu.sync_copy(img_h.at[pl.ds(base, CH)], vimg)
            pltpu.sync_copy(hkl_h.at[pl.ds(base, CH), :], vhkl)
            pltpu.sync_copy(eps_h.at[pl.ds(base, CH), :], veps)

            def _mkidx(j, carry2):
                r = j * 16 + iota
                h0 = plsc.load_gather(vhkl, [r, jnp.full((16,), 0, jnp.int32)])
                h1 = plsc.load_gather(vhkl, [r, jnp.full((16,), 1, jnp.int32)])
                h2 = plsc.load_gather(vhkl, [r, jnp.full((16,), 2, jnp.int32)])
                vidx[pl.ds(j * 16, 16)] = (h0 * HMAX + h1) * HMAX + h2
                return carry2
            lax.fori_loop(0, CH // 16, _mkidx, None)

            cps = []
            for b in range(12):
                cps.append(pltpu.async_copy(
                    zt_h.at[vidx.at[pl.ds(b * 128, 128)]],
                    fbuf.at[pl.ds(b * 128, 128)], sem))
            cps.append(pltpu.async_copy(
                zt_h.at[vidx.at[pl.ds(1536, 64)]],
                fbuf.at[pl.ds(1536, 64)], sem))
            for cp in cps:
                cp.wait()

            def _comp(j, carry2):
                s16 = pl.ds(j * 16, 16)
                lo = vloc[s16]
                si = vsig[s16]
                io = vio[s16]
                sg = vsg[s16]
                c4 = vc4[s16]
                idv = vimg[s16]
                t = io * iv
                den = sg * iv
                accip = zeros16
                accd2 = zeros16
                for mc in range(MC):
                    ep = plsc.load_gather(veps, [j * 16 + iota, jnp.full((16,), mc, jnp.int32)])
                    fv = plsc.load_gather(
                        fbuf, [j * 16 + iota, jnp.full((16,), mc, jnp.int32)])
                    sc = jnp.exp(lo + si * ep)
                    ip = fv * fv * sc
                    accip = accip + ip
                    dd = ip - t
                    accd2 = accd2 + dd * dd
                vipred[s16] = accip * 0.25
                llv = -0.5 * accd2 / (den * den) + c4
                s = plsc.cumsum(llv)
                id_nx = _take16(idv, jnp.minimum(iota + 1, 15))
                id_pv = _take16(idv, jnp.maximum(iota - 1, 0))
                s_pv = _take16(s, jnp.maximum(iota - 1, 0))
                endm = (idv != id_nx) | (iota == 15)
                stm = (idv != id_pv) & (iota > 0)
                fio = (iota + 1).astype(jnp.float32)
                plsc.addupdate_scatter(acc_ll, [idv], s, mask=endm)
                plsc.addupdate_scatter(acc_ll, [idv], -s_pv, mask=stm)
                plsc.addupdate_scatter(acc_cn, [idv], fio, mask=endm)
                plsc.addupdate_scatter(acc_cn, [idv], -(fio - 1.0), mask=stm)
                return carry2
            lax.fori_loop(0, CH // 16, _comp, None)

            pltpu.sync_copy(vipred, ipred_h.at[pl.ds(base, CH)])
        return carry
    lax.fori_loop(0, KMAX, _chunk, None)

    pltpu.sync_copy(acc_ll, llp_h.at[wid])
    pltpu.sync_copy(acc_cn, cnp_h.at[wid])


def kernel(metadata, iobs, sigiobs, w_loc, b_loc, w_sig, b_sig,
           q_loc, q_scale_raw, eps_scale, eps_z, image_id, rasu_id, hkl_in):
    f32 = jnp.float32

    msum, msq, isq = pl.pallas_call(
        _stats,
        grid=(GRID,),
        in_specs=[pl.BlockSpec((BN, DM), lambda i: (i, 0)),
                  pl.BlockSpec((1, 1, BN), lambda i: (i, 0, 0))],
        out_specs=[pl.BlockSpec((1, DM), lambda i: (0, 0)),
                   pl.BlockSpec((1, DM), lambda i: (0, 0)),
                   pl.BlockSpec((1, 1), lambda i: (0, 0))],
        out_shape=[jax.ShapeDtypeStruct((1, DM), f32),
                   jax.ShapeDtypeStruct((1, DM), f32),
                   jax.ShapeDtypeStruct((1, 1), f32)],
    )(metadata, iobs.reshape(GRID, 1, BN))

    mu = msum[0] / N
    var = msq[0] / N - mu * mu
    std = jnp.sqrt(var) + 1e-6
    i_std = jnp.sqrt(isq[0, 0] / N) + 1e-6
    inv_istd = 1.0 / i_std

    wl_adj = (w_loc / std).reshape(1, DM)
    ws_adj = (w_sig / std).reshape(1, DM)
    b_loc_adj = b_loc - jnp.sum(mu * w_loc / std)
    b_sig_adj = b_sig - jnp.sum(mu * w_sig / std)
    la = jnp.log(i_std) - 0.5 * LOG2PI
    cst = jnp.stack([b_loc_adj, b_sig_adj, la, jnp.zeros((), f32)]).reshape(1, 4)

    zt, klacc = pl.pallas_call(
        _post,
        grid=(NU // 256,),
        in_specs=[pl.BlockSpec((256, 1), lambda i: (i, 0)),
                  pl.BlockSpec((256, 1), lambda i: (i, 0)),
                  pl.BlockSpec((256, MC), lambda i: (i, 0))],
        out_specs=[pl.BlockSpec((256, 16), lambda i: (i, 0)),
                   pl.BlockSpec((1, 1), lambda i: (0, 0))],
        out_shape=[jax.ShapeDtypeStruct((NU, 16), f32),
                   jax.ShapeDtypeStruct((1, 1), f32)],
    )(q_loc.reshape(NU, 1), q_scale_raw.reshape(NU, 1), eps_z.T.reshape(NU, MC))

    loc2, sig2, c42, sklacc = pl.pallas_call(
        _lin,
        grid=(GRID,),
        in_specs=[pl.BlockSpec((BN, DM), lambda i: (i, 0)),
                  pl.BlockSpec((1, 1, BN), lambda i: (i, 0, 0)),
                  pl.BlockSpec((1, DM), lambda i: (0, 0)),
                  pl.BlockSpec((1, DM), lambda i: (0, 0)),
                  pl.BlockSpec((1, 4), lambda i: (0, 0))],
        out_specs=[pl.BlockSpec((1, 1, BN), lambda i: (i, 0, 0)),
                   pl.BlockSpec((1, 1, BN), lambda i: (i, 0, 0)),
                   pl.BlockSpec((1, 1, BN), lambda i: (i, 0, 0)),
                   pl.BlockSpec((1, 1), lambda i: (0, 0))],
        out_shape=[jax.ShapeDtypeStruct((GRID, 1, BN), f32),
                   jax.ShapeDtypeStruct((GRID, 1, BN), f32),
                   jax.ShapeDtypeStruct((GRID, 1, BN), f32),
                   jax.ShapeDtypeStruct((1, 1), f32)],
    )(metadata, sigiobs.reshape(GRID, 1, BN), wl_adj, ws_adj, cst)

    scal16 = jnp.full((16,), inv_istd, f32)

    mesh = plsc.VectorSubcoreMesh(core_axis_name="c", subcore_axis_name="s")
    sc = functools.partial(
        pl.kernel,
        mesh=mesh,
        compiler_params=pltpu.CompilerParams(
            needs_layout_passes=False, use_tc_tiling_on_sc=False),
        out_type=[jax.ShapeDtypeStruct((N,), f32),
                  jax.ShapeDtypeStruct((NW, NIMG), f32),
                  jax.ShapeDtypeStruct((NW, NIMG), f32)],
        scratch_types=[
            pltpu.VMEM((CH,), f32),
            pltpu.VMEM((CH,), f32),
            pltpu.VMEM((CH,), f32),
            pltpu.VMEM((CH,), f32),
            pltpu.VMEM((CH,), f32),
            pltpu.VMEM((CH,), jnp.int32),
            pltpu.VMEM((CH, 3), jnp.int32),
            pltpu.VMEM((CH, MC), f32),
            pltpu.VMEM((CH,), jnp.int32),
            pltpu.VMEM((CH, 16), f32),
            pltpu.VMEM((CH,), f32),
            pltpu.VMEM((NIMG,), f32),
            pltpu.VMEM((NIMG,), f32),
            pltpu.VMEM((16,), f32),
            pltpu.SemaphoreType.DMA,
        ],
    )(_sc_body)
    ipred, llp, cnp = sc(
        loc2.reshape(N), sig2.reshape(N), iobs, sigiobs, c42.reshape(N),
        image_id.astype(jnp.int32), hkl_in.astype(jnp.int32),
        eps_scale, zt, scal16)

    nll2 = pl.pallas_call(
        _comb,
        grid=(1,),
        in_specs=[pl.BlockSpec((NW, NIMG), lambda i: (0, 0)),
                  pl.BlockSpec((NW, NIMG), lambda i: (0, 0))],
        out_specs=pl.BlockSpec((1, NIMG), lambda i: (0, 0)),
        out_shape=jax.ShapeDtypeStruct((1, NIMG), f32),
    )(llp, cnp)

    kl_div = klacc[0, 0] / (NU * MC)
    scale_kl_div = sklacc[0, 0] / N
    return (ipred, nll2.reshape(NIMG), kl_div, scale_kl_div)


# speedup vs baseline: 1.1084x; 1.1084x over previous
import functools
import math

import jax
import jax.numpy as jnp
from jax import lax
from jax.experimental import pallas as pl
from jax.experimental.pallas import tpu as pltpu
from jax.experimental.pallas import tpu_sc as plsc

N = 1_000_000
NIMG = 10_000
DM = 10
MC = 4
HMAX = 32
NU = HMAX ** 3
LOG2PI = float(math.log(2.0 * math.pi))

BN = 8000
GRID = N // BN
CH = 1600
NCHUNK = N // CH
NW = 32
KMAX = -(-NCHUNK // NW)


def _stats(md_ref, io_ref, msum_ref, msq_ref, isq_ref):
    @pl.when(pl.program_id(0) == 0)
    def _():
        msum_ref[...] = jnp.zeros_like(msum_ref)
        msq_ref[...] = jnp.zeros_like(msq_ref)
        isq_ref[...] = jnp.zeros_like(isq_ref)

    m = md_ref[...]
    msum_ref[...] += m.sum(axis=0, keepdims=True)
    msq_ref[...] += (m * m).sum(axis=0, keepdims=True)
    io = io_ref[...]
    isq_ref[...] += jnp.sum(io * io).reshape(1, 1)


def _post(ql_ref, qsr_ref, ep_ref, zt_ref, kl_ref):
    @pl.when(pl.program_id(0) == 0)
    def _():
        kl_ref[...] = jnp.zeros_like(kl_ref)

    ql = ql_ref[...]
    qs = jax.nn.softplus(qsr_ref[...]) + 1e-6
    ep = ep_ref[...]
    z = ql + qs * ep
    zt_ref[...] = jnp.concatenate([z, jnp.zeros((z.shape[0], 12), jnp.float32)], axis=1)
    kl_ref[...] += jnp.sum(0.5 * z * z - 0.5 * ep * ep - jnp.log(qs)).reshape(1, 1)


def _lin(md_ref, sio_ref, wl_ref, ws_ref, cst_ref,
         loc_ref, sig_ref, c4_ref, skl_ref):
    @pl.when(pl.program_id(0) == 0)
    def _():
        skl_ref[...] = jnp.zeros_like(skl_ref)

    m = md_ref[...]
    wl = wl_ref[...]
    ws = ws_ref[...]
    cst = cst_ref[...]
    loc = (m * wl).sum(axis=1) + cst[0, 0]
    sig = jax.nn.softplus((m * ws).sum(axis=1) + cst[0, 1]) + 1e-6
    loc_ref[...] = loc.reshape(1, 1, BN)
    sig_ref[...] = sig.reshape(1, 1, BN)
    sio = sio_ref[...]
    c4_ref[...] = 4.0 * (cst[0, 2] - jnp.log(sio))
    skl_ref[...] += jnp.sum(0.5 * (loc * loc + sig * sig - 2.0 * jnp.log(sig) - 1.0)).reshape(1, 1)


def _comb(llp_ref, cnp_ref, out_ref):
    s = llp_ref[...].sum(axis=0)
    c = jnp.maximum(cnp_ref[...].sum(axis=0), 1.0)
    out_ref[...] = (-(s / c) * (1.0 / MC)).reshape(1, NIMG)


def _take16(x, idx):
    dnums = lax.GatherDimensionNumbers(
        offset_dims=(), collapsed_slice_dims=(0,), start_index_map=(0,))
    return lax.gather(x, idx[:, None], dnums, (1,),
                      mode=lax.GatherScatterMode.PROMISE_IN_BOUNDS)


def _sc_body(loc_h, sig_h, io_h, sg_h, c4_h, img_h, hkl_h, eps_h, zt_h, scal_h,
             ipred_h, llp_h, cnp_h,
             vloc, vsig, vio, vsg, vc4, vimg, vhkl, veps, vidx, fbuf, vipred,
             acc_ll, acc_cn, scalv, sem):
    wid = lax.axis_index("s") * 2 + lax.axis_index("c")
    iota = lax.iota(jnp.int32, 16)
    zeros16 = jnp.zeros((16,), jnp.float32)

    def _zinit(i, carry):
        acc_ll[pl.ds(i * 16, 16)] = zeros16
        acc_cn[pl.ds(i * 16, 16)] = zeros16
        return carry
    lax.fori_loop(0, NIMG // 16, _zinit, None)

    pltpu.sync_copy(scal_h, scalv)
    iv = scalv[...]

    def _chunk(k, carry):
        c = wid + NW * k

        @pl.when(c < NCHUNK)
        def _():
            base = c * CH
            pltpu.sync_copy(loc_h.at[pl.ds(base, CH)], vloc)
            pltpu.sync_copy(sig_h.at[pl.ds(base, CH)], vsig)
            pltpu.sync_copy(io_h.at[pl.ds(base, CH)], vio)
            pltpu.sync_copy(sg_h.at[pl.ds(base, CH)], vsg)
            pltpu.sync_copy(c4_h.at[pl.ds(base, CH)], vc4)
            pltpu.sync_copy(img_h.at[pl.ds(base, CH)], vimg)
            pltpu.sync_copy(hkl_h.at[pl.ds(base * 3, CH * 3)], vhkl)
            pltpu.sync_copy(eps_h.at[pl.ds(base * 4, CH * 4)], veps)

            def _mkidx(j, carry2):
                b3 = j * 48
                h0 = plsc.load_gather(vhkl, [b3 + iota * 3])
                h1 = plsc.load_gather(vhkl, [b3 + iota * 3 + 1])
                h2 = plsc.load_gather(vhkl, [b3 + iota * 3 + 2])
                vidx[pl.ds(j * 16, 16)] = (h0 * HMAX + h1) * HMAX + h2
                return carry2
            lax.fori_loop(0, CH // 16, _mkidx, None)

            cps = []
            for b in range(12):
                cps.append(pltpu.async_copy(
                    zt_h.at[vidx.at[pl.ds(b * 128, 128)]],
                    fbuf.at[pl.ds(b * 128, 128)], sem))
            cps.append(pltpu.async_copy(
                zt_h.at[vidx.at[pl.ds(1536, 64)]],
                fbuf.at[pl.ds(1536, 64)], sem))
            for cp in cps:
                cp.wait()

            def _comp(j, carry2):
                s16 = pl.ds(j * 16, 16)
                lo = vloc[s16]
                si = vsig[s16]
                io = vio[s16]
                sg = vsg[s16]
                c4 = vc4[s16]
                idv = vimg[s16]
                t = io * iv
                den = sg * iv
                accip = zeros16
                accd2 = zeros16
                for mc in range(MC):
                    ep = plsc.load_gather(veps, [j * 64 + iota * 4 + mc])
                    fv = plsc.load_gather(
                        fbuf, [j * 16 + iota, jnp.full((16,), mc, jnp.int32)])
                    sc = jnp.exp(lo + si * ep)
                    ip = fv * fv * sc
                    accip = accip + ip
                    dd = ip - t
                    accd2 = accd2 + dd * dd
                vipred[s16] = accip * 0.25
                llv = -0.5 * accd2 / (den * den) + c4
                s = plsc.cumsum(llv)
                id_nx = _take16(idv, jnp.minimum(iota + 1, 15))
                id_pv = _take16(idv, jnp.maximum(iota - 1, 0))
                s_pv = _take16(s, jnp.maximum(iota - 1, 0))
                endm = (idv != id_nx) | (iota == 15)
                stm = (idv != id_pv) & (iota > 0)
                fio = (iota + 1).astype(jnp.float32)
                plsc.addupdate_scatter(acc_ll, [idv], s, mask=endm)
                plsc.addupdate_scatter(acc_ll, [idv], -s_pv, mask=stm)
                plsc.addupdate_scatter(acc_cn, [idv], fio, mask=endm)
                plsc.addupdate_scatter(acc_cn, [idv], -(fio - 1.0), mask=stm)
                return carry2
            lax.fori_loop(0, CH // 16, _comp, None)

            pltpu.sync_copy(vipred, ipred_h.at[pl.ds(base, CH)])
        return carry
    lax.fori_loop(0, KMAX, _chunk, None)

    pltpu.sync_copy(acc_ll, llp_h.at[wid])
    pltpu.sync_copy(acc_cn, cnp_h.at[wid])


def kernel(metadata, iobs, sigiobs, w_loc, b_loc, w_sig, b_sig,
           q_loc, q_scale_raw, eps_scale, eps_z, image_id, rasu_id, hkl_in):
    f32 = jnp.float32

    msum, msq, isq = pl.pallas_call(
        _stats,
        grid=(GRID,),
        in_specs=[pl.BlockSpec((BN, DM), lambda i: (i, 0)),
                  pl.BlockSpec((1, 1, BN), lambda i: (i, 0, 0))],
        out_specs=[pl.BlockSpec((1, DM), lambda i: (0, 0)),
                   pl.BlockSpec((1, DM), lambda i: (0, 0)),
                   pl.BlockSpec((1, 1), lambda i: (0, 0))],
        out_shape=[jax.ShapeDtypeStruct((1, DM), f32),
                   jax.ShapeDtypeStruct((1, DM), f32),
                   jax.ShapeDtypeStruct((1, 1), f32)],
    )(metadata, iobs.reshape(GRID, 1, BN))

    mu = msum[0] / N
    var = msq[0] / N - mu * mu
    std = jnp.sqrt(var) + 1e-6
    i_std = jnp.sqrt(isq[0, 0] / N) + 1e-6
    inv_istd = 1.0 / i_std

    wl_adj = (w_loc / std).reshape(1, DM)
    ws_adj = (w_sig / std).reshape(1, DM)
    b_loc_adj = b_loc - jnp.sum(mu * w_loc / std)
    b_sig_adj = b_sig - jnp.sum(mu * w_sig / std)
    la = jnp.log(i_std) - 0.5 * LOG2PI
    cst = jnp.stack([b_loc_adj, b_sig_adj, la, jnp.zeros((), f32)]).reshape(1, 4)

    zt, klacc = pl.pallas_call(
        _post,
        grid=(NU // 256,),
        in_specs=[pl.BlockSpec((256, 1), lambda i: (i, 0)),
                  pl.BlockSpec((256, 1), lambda i: (i, 0)),
                  pl.BlockSpec((256, MC), lambda i: (i, 0))],
        out_specs=[pl.BlockSpec((256, 16), lambda i: (i, 0)),
                   pl.BlockSpec((1, 1), lambda i: (0, 0))],
        out_shape=[jax.ShapeDtypeStruct((NU, 16), f32),
                   jax.ShapeDtypeStruct((1, 1), f32)],
    )(q_loc.reshape(NU, 1), q_scale_raw.reshape(NU, 1), eps_z.T.reshape(NU, MC))

    loc2, sig2, c42, sklacc = pl.pallas_call(
        _lin,
        grid=(GRID,),
        in_specs=[pl.BlockSpec((BN, DM), lambda i: (i, 0)),
                  pl.BlockSpec((1, 1, BN), lambda i: (i, 0, 0)),
                  pl.BlockSpec((1, DM), lambda i: (0, 0)),
                  pl.BlockSpec((1, DM), lambda i: (0, 0)),
                  pl.BlockSpec((1, 4), lambda i: (0, 0))],
        out_specs=[pl.BlockSpec((1, 1, BN), lambda i: (i, 0, 0)),
                   pl.BlockSpec((1, 1, BN), lambda i: (i, 0, 0)),
                   pl.BlockSpec((1, 1, BN), lambda i: (i, 0, 0)),
                   pl.BlockSpec((1, 1), lambda i: (0, 0))],
        out_shape=[jax.ShapeDtypeStruct((GRID, 1, BN), f32),
                   jax.ShapeDtypeStruct((GRID, 1, BN), f32),
                   jax.ShapeDtypeStruct((GRID, 1, BN), f32),
                   jax.ShapeDtypeStruct((1, 1), f32)],
    )(metadata, sigiobs.reshape(GRID, 1, BN), wl_adj, ws_adj, cst)

    scal16 = jnp.full((16,), inv_istd, f32)

    mesh = plsc.VectorSubcoreMesh(core_axis_name="c", subcore_axis_name="s")
    sc = functools.partial(
        pl.kernel,
        mesh=mesh,
        compiler_params=pltpu.CompilerParams(
            needs_layout_passes=False, use_tc_tiling_on_sc=False),
        out_type=[jax.ShapeDtypeStruct((N,), f32),
                  jax.ShapeDtypeStruct((NW, NIMG), f32),
                  jax.ShapeDtypeStruct((NW, NIMG), f32)],
        scratch_types=[
            pltpu.VMEM((CH,), f32),
            pltpu.VMEM((CH,), f32),
            pltpu.VMEM((CH,), f32),
            pltpu.VMEM((CH,), f32),
            pltpu.VMEM((CH,), f32),
            pltpu.VMEM((CH,), jnp.int32),
            pltpu.VMEM((CH * 3,), jnp.int32),
            pltpu.VMEM((CH * MC,), f32),
            pltpu.VMEM((CH,), jnp.int32),
            pltpu.VMEM((CH, 16), f32),
            pltpu.VMEM((CH,), f32),
            pltpu.VMEM((NIMG,), f32),
            pltpu.VMEM((NIMG,), f32),
            pltpu.VMEM((16,), f32),
            pltpu.SemaphoreType.DMA,
        ],
    )(_sc_body)
    ipred, llp, cnp = sc(
        loc2.reshape(N), sig2.reshape(N), iobs, sigiobs, c42.reshape(N),
        image_id.astype(jnp.int32), hkl_in.astype(jnp.int32).reshape(N * 3),
        eps_scale.reshape(N * MC), zt, scal16)

    nll2 = pl.pallas_call(
        _comb,
        grid=(1,),
        in_specs=[pl.BlockSpec((NW, NIMG), lambda i: (0, 0)),
                  pl.BlockSpec((NW, NIMG), lambda i: (0, 0))],
        out_specs=pl.BlockSpec((1, NIMG), lambda i: (0, 0)),
        out_shape=jax.ShapeDtypeStruct((1, NIMG), f32),
    )(llp, cnp)

    kl_div = klacc[0, 0] / (NU * MC)
    scale_kl_div = sklacc[0, 0] / N
    return (ipred, nll2.reshape(NIMG), kl_div, scale_kl_div)
